# Initial kernel scaffold; baseline (speedup 1.0000x reference)
import jax
import jax.numpy as jnp
from jax.experimental import pallas as pl


def kernel(x, edge_index0, edge_index1, Wg0, bg0, Wg1, bg1, Ws1, bs1, Ws2, Wp, bp):
    raise NotImplementedError("write your pallas kernel here")


# speedup vs baseline: 7.2700x; 7.2700x over previous
import functools
import jax
import jax.numpy as jnp
from jax import lax
from jax.experimental import pallas as pl
from jax.experimental.pallas import tpu as pltpu
from jax.experimental.pallas import tpu_sc as plsc

N = 10000
E = 320000
IN = 128
SEM_HID = 128
OUT = 64

NTILES = 16
ROWS_PER_TILE = 640
NPAD = NTILES * ROWS_PER_TILE
CHUNK = 128
STAGE = 32
STAGES = 5
CHUNKS = STAGE * STAGES
EPT = CHUNKS * CHUNK
EPAD = NTILES * EPT

f32 = jnp.float32
i32 = jnp.int32

_MESH = plsc.VectorSubcoreMesh(core_axis_name="c", subcore_axis_name="s")


@functools.partial(
    pl.kernel,
    out_type=[jax.ShapeDtypeStruct((NPAD,), f32)] * 4,
    mesh=_MESH,
    scratch_types=[
        pltpu.VMEM((CHUNKS, CHUNK), i32),
        pltpu.VMEM((CHUNK,), f32),
        pltpu.VMEM_SHARED((NPAD,), f32),
        pltpu.VMEM_SHARED((NPAD,), f32),
    ],
)
def _hist_kernel(src0, dst0, src1, dst1, z1, ds0, di0, ds1, di1,
                 idx_v, ones_v, hsrc, hdst):
    c = lax.axis_index("c")
    s = lax.axis_index("s")
    for i in range(CHUNK // 16):
        ones_v[pl.ds(i * 16, 16)] = jnp.ones((16,), f32)
    sl = pl.ds(s * ROWS_PER_TILE, ROWS_PER_TILE)
    pltpu.sync_copy(z1, hsrc.at[sl])
    pltpu.sync_copy(z1, hdst.at[sl])
    plsc.subcore_barrier()

    def accumulate(edges_hbm, hist):
        pltpu.sync_copy(edges_hbm.at[s], idx_v)

        def body(j, carry):
            pltpu.sync_copy(ones_v, hist.at[idx_v.at[j]], add=True)
            return carry
        lax.fori_loop(0, CHUNKS, body, 0)

    @pl.when(c == 0)
    def _():
        accumulate(src0, hsrc)
        accumulate(dst0, hdst)

    @pl.when(c == 1)
    def _():
        accumulate(src1, hsrc)
        accumulate(dst1, hdst)

    plsc.subcore_barrier()

    @pl.when(c == 0)
    def _():
        pltpu.sync_copy(hsrc.at[sl], ds0.at[sl])
        pltpu.sync_copy(hdst.at[sl], di0.at[sl])

    @pl.when(c == 1)
    def _():
        pltpu.sync_copy(hsrc.at[sl], ds1.at[sl])
        pltpu.sync_copy(hdst.at[sl], di1.at[sl])


@functools.partial(
    pl.kernel,
    out_type=[jax.ShapeDtypeStruct((NPAD, IN), f32)] * 2,
    mesh=_MESH,
    scratch_types=[
        pltpu.VMEM((STAGE, CHUNK), i32),
        pltpu.VMEM((STAGE, CHUNK), i32),
        pltpu.VMEM((CHUNK, IN), f32),
        pltpu.VMEM_SHARED((NPAD, IN), f32),
    ],
)
def _scatter_kernel(xs0, xs1, src0, dst0, src1, dst1, z2, y0, y1,
                    src_v, dst_v, rows_v, ys):
    c = lax.axis_index("c")
    s = lax.axis_index("s")
    sl = pl.ds(s * ROWS_PER_TILE, ROWS_PER_TILE)
    pltpu.sync_copy(z2, ys.at[sl])
    plsc.subcore_barrier()

    def run(xs_hbm, src_hbm, dst_hbm):
        def stage_body(st, carry):
            pltpu.sync_copy(src_hbm.at[s, pl.ds(st * STAGE, STAGE)], src_v)
            pltpu.sync_copy(dst_hbm.at[s, pl.ds(st * STAGE, STAGE)], dst_v)

            def body(j, carry2):
                pltpu.sync_copy(xs_hbm.at[src_v.at[j]], rows_v)
                pltpu.sync_copy(rows_v, ys.at[dst_v.at[j]], add=True)
                return carry2
            lax.fori_loop(0, STAGE, body, 0)
            return carry
        lax.fori_loop(0, STAGES, stage_body, 0)

    @pl.when(c == 0)
    def _():
        run(xs0, src0, dst0)

    @pl.when(c == 1)
    def _():
        run(xs1, src1, dst1)

    plsc.subcore_barrier()

    @pl.when(c == 0)
    def _():
        pltpu.sync_copy(ys.at[sl], y0.at[sl])

    @pl.when(c == 1)
    def _():
        pltpu.sync_copy(ys.at[sl], y1.at[sl])


def _wfold_body(Wg0, Wg1, Ws1, bs1, Wp, bg0, bg1,
                M0, M1, k0, k1, G0, G1, g0, g1):
    M0[...] = jnp.dot(Wg0[...], Ws1[...], preferred_element_type=f32)
    M1[...] = jnp.dot(Wg1[...], Ws1[...], preferred_element_type=f32)
    k0[...] = jnp.dot(bg0[...], Ws1[...], preferred_element_type=f32) + bs1[...]
    k1[...] = jnp.dot(bg1[...], Ws1[...], preferred_element_type=f32) + bs1[...]
    G0[...] = jnp.dot(Wg0[...], Wp[...], preferred_element_type=f32)
    G1[...] = jnp.dot(Wg1[...], Wp[...], preferred_element_type=f32)
    g0[...] = jnp.dot(bg0[...], Wp[...], preferred_element_type=f32)
    g1[...] = jnp.dot(bg1[...], Wp[...], preferred_element_type=f32)


def _scale_body(x_ref, d0_ref, d1_ref, xs0_ref, xs1_ref):
    x = x_ref[...]
    n0 = lax.rsqrt(jnp.maximum(d0_ref[...], 1.0))
    n1 = lax.rsqrt(jnp.maximum(d1_ref[...], 1.0))
    xs0_ref[...] = x * n0
    xs1_ref[...] = x * n1


_RB = 256
_NB = NPAD // _RB


def _score_body(y0_ref, y1_ref, d0_ref, d1_ref, M0_ref, M1_ref,
                k0_ref, k1_ref, w2_ref, sum_ref):
    i = pl.program_id(0)
    rows = lax.broadcasted_iota(i32, (_RB, 1), 0) + i * _RB
    mask = rows < N

    def part(y_ref, d_ref, M_ref, k_ref):
        yb = y_ref[...] * lax.rsqrt(jnp.maximum(d_ref[...], 1.0))
        a = jnp.tanh(jnp.dot(yb, M_ref[...], preferred_element_type=f32)
                     + k_ref[...])
        t = jnp.sum(a * w2_ref[...], axis=1, keepdims=True)
        return jnp.sum(jnp.where(mask, t, 0.0))

    s0 = part(y0_ref, d0_ref, M0_ref, k0_ref)
    s1 = part(y1_ref, d1_ref, M1_ref, k1_ref)

    row = lax.broadcasted_iota(i32, (8, 128), 0)
    lane = lax.broadcasted_iota(i32, (8, 128), 1)
    contrib = (jnp.where((row == 0) & (lane == 0), s0, 0.0)
               + jnp.where((row == 1) & (lane == 0), s1, 0.0))

    @pl.when(i == 0)
    def _():
        sum_ref[...] = jnp.zeros((8, 128), f32)

    sum_ref[...] += contrib


def _out_body(sum_ref, y0_ref, y1_ref, d0_ref, d1_ref, G0_ref, G1_ref,
              g0_ref, g1_ref, bp_ref, o_ref):
    srow = sum_ref[...]
    row = lax.broadcasted_iota(i32, (8, 128), 0)
    lane = lax.broadcasted_iota(i32, (8, 128), 1)
    w0 = jnp.sum(jnp.where((row == 0) & (lane == 0), srow, 0.0)) / N
    w1 = jnp.sum(jnp.where((row == 1) & (lane == 0), srow, 0.0)) / N
    m = jnp.maximum(w0, w1)
    e0 = jnp.exp(w0 - m)
    e1 = jnp.exp(w1 - m)
    b0 = e0 / (e0 + e1)
    b1 = e1 / (e0 + e1)
    yb0 = y0_ref[...] * lax.rsqrt(jnp.maximum(d0_ref[...], 1.0))
    yb1 = y1_ref[...] * lax.rsqrt(jnp.maximum(d1_ref[...], 1.0))
    p0 = jnp.dot(yb0, G0_ref[...], preferred_element_type=f32) + g0_ref[...]
    p1 = jnp.dot(yb1, G1_ref[...], preferred_element_type=f32) + g1_ref[...]
    o_ref[...] = b0 * p0 + b1 * p1 + bp_ref[...]


def _pad_edges(idx):
    pad = jnp.full((EPAD - E,), N, dtype=i32)
    return jnp.concatenate([idx, pad]).reshape(NTILES, CHUNKS, CHUNK)


def kernel(x, edge_index0, edge_index1, Wg0, bg0, Wg1, bg1,
           Ws1, bs1, Ws2, Wp, bp):
    src0 = _pad_edges(edge_index0[0])
    dst0 = _pad_edges(edge_index0[1])
    src1 = _pad_edges(edge_index1[0])
    dst1 = _pad_edges(edge_index1[1])
    x_pad = jnp.pad(x, ((0, NPAD - N), (0, 0)))
    z1 = jnp.zeros((ROWS_PER_TILE,), f32)
    z2 = jnp.zeros((ROWS_PER_TILE, IN), f32)

    ds0, di0, ds1, di1 = _hist_kernel(src0, dst0, src1, dst1, z1)

    M0, M1, k0, k1, G0, G1, g0, g1 = pl.pallas_call(
        _wfold_body,
        out_shape=[
            jax.ShapeDtypeStruct((IN, SEM_HID), f32),
            jax.ShapeDtypeStruct((IN, SEM_HID), f32),
            jax.ShapeDtypeStruct((1, SEM_HID), f32),
            jax.ShapeDtypeStruct((1, SEM_HID), f32),
            jax.ShapeDtypeStruct((IN, OUT), f32),
            jax.ShapeDtypeStruct((IN, OUT), f32),
            jax.ShapeDtypeStruct((1, OUT), f32),
            jax.ShapeDtypeStruct((1, OUT), f32),
        ],
    )(Wg0, Wg1, Ws1, bs1.reshape(1, SEM_HID), Wp,
      bg0.reshape(1, -1), bg1.reshape(1, -1))

    xs0, xs1 = pl.pallas_call(
        _scale_body,
        grid=(_NB,),
        in_specs=[
            pl.BlockSpec((_RB, IN), lambda i: (i, 0)),
            pl.BlockSpec((_RB, 1), lambda i: (i, 0)),
            pl.BlockSpec((_RB, 1), lambda i: (i, 0)),
        ],
        out_specs=[
            pl.BlockSpec((_RB, IN), lambda i: (i, 0)),
            pl.BlockSpec((_RB, IN), lambda i: (i, 0)),
        ],
        out_shape=[jax.ShapeDtypeStruct((NPAD, IN), f32)] * 2,
    )(x_pad, ds0.reshape(NPAD, 1), ds1.reshape(NPAD, 1))

    y0, y1 = _scatter_kernel(xs0, xs1, src0, dst0, src1, dst1, z2)

    di0c = di0.reshape(NPAD, 1)
    di1c = di1.reshape(NPAD, 1)
    w2r = Ws2.reshape(1, SEM_HID)

    full = lambda i: (0, 0)
    sums = pl.pallas_call(
        _score_body,
        grid=(_NB,),
        in_specs=[
            pl.BlockSpec((_RB, IN), lambda i: (i, 0)),
            pl.BlockSpec((_RB, IN), lambda i: (i, 0)),
            pl.BlockSpec((_RB, 1), lambda i: (i, 0)),
            pl.BlockSpec((_RB, 1), lambda i: (i, 0)),
            pl.BlockSpec((IN, SEM_HID), full),
            pl.BlockSpec((IN, SEM_HID), full),
            pl.BlockSpec((1, SEM_HID), full),
            pl.BlockSpec((1, SEM_HID), full),
            pl.BlockSpec((1, SEM_HID), full),
        ],
        out_specs=pl.BlockSpec((8, 128), full),
        out_shape=jax.ShapeDtypeStruct((8, 128), f32),
    )(y0, y1, di0c, di1c, M0, M1, k0, k1, w2r)

    outp = pl.pallas_call(
        _out_body,
        grid=(_NB,),
        in_specs=[
            pl.BlockSpec((8, 128), full),
            pl.BlockSpec((_RB, IN), lambda i: (i, 0)),
            pl.BlockSpec((_RB, IN), lambda i: (i, 0)),
            pl.BlockSpec((_RB, 1), lambda i: (i, 0)),
            pl.BlockSpec((_RB, 1), lambda i: (i, 0)),
            pl.BlockSpec((IN, OUT), full),
            pl.BlockSpec((IN, OUT), full),
            pl.BlockSpec((1, OUT), full),
            pl.BlockSpec((1, OUT), full),
            pl.BlockSpec((1, OUT), full),
        ],
        out_specs=pl.BlockSpec((_RB, OUT), lambda i: (i, 0)),
        out_shape=jax.ShapeDtypeStruct((NPAD, OUT), f32),
    )(sums, y0, y1, di0c, di1c, G0, G1, g0, g1, bp.reshape(1, OUT))

    return outp[:N]


# speedup vs baseline: 8.4084x; 1.1566x over previous
import functools
import jax
import jax.numpy as jnp
from jax import lax
from jax.experimental import pallas as pl
from jax.experimental.pallas import tpu as pltpu
from jax.experimental.pallas import tpu_sc as plsc

N = 10000
E = 320000
IN = 128
SEM_HID = 128
OUT = 64

NTILES = 16
ROWS_PER_TILE = 640
NPAD = NTILES * ROWS_PER_TILE
CHUNK = 128
STAGE = 32
STAGES = 5
CHUNKS = STAGE * STAGES
EPT = CHUNKS * CHUNK
EPAD = NTILES * EPT

f32 = jnp.float32
i32 = jnp.int32

_MESH = plsc.VectorSubcoreMesh(core_axis_name="c", subcore_axis_name="s")


@functools.partial(
    pl.kernel,
    out_type=[jax.ShapeDtypeStruct((NPAD,), f32)] * 4,
    mesh=_MESH,
    scratch_types=[
        pltpu.VMEM((CHUNKS, CHUNK), i32),
        pltpu.VMEM((CHUNK,), f32),
        pltpu.VMEM_SHARED((NPAD,), f32),
        pltpu.VMEM_SHARED((NPAD,), f32),
    ],
)
def _hist_kernel(src0, dst0, src1, dst1, z1, ds0, di0, ds1, di1,
                 idx_v, ones_v, hsrc, hdst):
    c = lax.axis_index("c")
    s = lax.axis_index("s")
    for i in range(CHUNK // 16):
        ones_v[pl.ds(i * 16, 16)] = jnp.ones((16,), f32)
    sl = pl.ds(s * ROWS_PER_TILE, ROWS_PER_TILE)
    pltpu.sync_copy(z1, hsrc.at[sl])
    pltpu.sync_copy(z1, hdst.at[sl])
    plsc.subcore_barrier()

    def accumulate(edges_hbm, hist):
        pltpu.sync_copy(edges_hbm.at[s], idx_v)

        def body(j, carry):
            pltpu.sync_copy(ones_v, hist.at[idx_v.at[j]], add=True)
            return carry
        lax.fori_loop(0, CHUNKS, body, 0)

    @pl.when(c == 0)
    def _():
        accumulate(src0, hsrc)
        accumulate(dst0, hdst)

    @pl.when(c == 1)
    def _():
        accumulate(src1, hsrc)
        accumulate(dst1, hdst)

    plsc.subcore_barrier()

    @pl.when(c == 0)
    def _():
        pltpu.sync_copy(hsrc.at[sl], ds0.at[sl])
        pltpu.sync_copy(hdst.at[sl], di0.at[sl])

    @pl.when(c == 1)
    def _():
        pltpu.sync_copy(hsrc.at[sl], ds1.at[sl])
        pltpu.sync_copy(hdst.at[sl], di1.at[sl])


@functools.partial(
    pl.kernel,
    out_type=[jax.ShapeDtypeStruct((NPAD, IN), f32)] * 2,
    mesh=_MESH,
    scratch_types=[
        pltpu.VMEM((STAGE, CHUNK), i32),
        pltpu.VMEM((STAGE, CHUNK), i32),
        pltpu.VMEM((CHUNK, IN), f32),
        pltpu.VMEM((CHUNK, IN), f32),
        pltpu.VMEM_SHARED((NPAD, IN), f32),
        pltpu.SemaphoreType.DMA,
        pltpu.SemaphoreType.DMA,
    ],
)
def _scatter_kernel(xs0, xs1, src0, dst0, src1, dst1, z2, y0, y1,
                    src_v, dst_v, buf0, buf1, ys, sem0, sem1):
    c = lax.axis_index("c")
    s = lax.axis_index("s")
    sl = pl.ds(s * ROWS_PER_TILE, ROWS_PER_TILE)
    pltpu.sync_copy(z2, ys.at[sl])
    plsc.subcore_barrier()

    def run(xs_hbm, src_hbm, dst_hbm):
        def stage_body(st, carry):
            pltpu.sync_copy(src_hbm.at[s, pl.ds(st * STAGE, STAGE)], src_v)
            pltpu.sync_copy(dst_hbm.at[s, pl.ds(st * STAGE, STAGE)], dst_v)
            pltpu.async_copy(xs_hbm.at[src_v.at[0]], buf0, sem0)

            def body(i, carry2):
                pltpu.async_copy(xs_hbm.at[src_v.at[2 * i + 1]], buf1, sem1)
                pltpu.make_async_copy(xs_hbm.at[src_v.at[0]], buf0, sem0).wait()
                pltpu.sync_copy(buf0, ys.at[dst_v.at[2 * i]], add=True)

                @pl.when(i < STAGE // 2 - 1)
                def _():
                    pltpu.async_copy(xs_hbm.at[src_v.at[2 * i + 2]], buf0, sem0)

                pltpu.make_async_copy(xs_hbm.at[src_v.at[0]], buf1, sem1).wait()
                pltpu.sync_copy(buf1, ys.at[dst_v.at[2 * i + 1]], add=True)
                return carry2
            lax.fori_loop(0, STAGE // 2, body, 0)
            return carry
        lax.fori_loop(0, STAGES, stage_body, 0)

    @pl.when(c == 0)
    def _():
        run(xs0, src0, dst0)

    @pl.when(c == 1)
    def _():
        run(xs1, src1, dst1)

    plsc.subcore_barrier()

    @pl.when(c == 0)
    def _():
        pltpu.sync_copy(ys.at[sl], y0.at[sl])

    @pl.when(c == 1)
    def _():
        pltpu.sync_copy(ys.at[sl], y1.at[sl])


def _wfold_body(Wg0, Wg1, Ws1, bs1, Wp, bg0, bg1,
                M0, M1, k0, k1, G0, G1, g0, g1):
    M0[...] = jnp.dot(Wg0[...], Ws1[...], preferred_element_type=f32)
    M1[...] = jnp.dot(Wg1[...], Ws1[...], preferred_element_type=f32)
    k0[...] = jnp.dot(bg0[...], Ws1[...], preferred_element_type=f32) + bs1[...]
    k1[...] = jnp.dot(bg1[...], Ws1[...], preferred_element_type=f32) + bs1[...]
    G0[...] = jnp.dot(Wg0[...], Wp[...], preferred_element_type=f32)
    G1[...] = jnp.dot(Wg1[...], Wp[...], preferred_element_type=f32)
    g0[...] = jnp.dot(bg0[...], Wp[...], preferred_element_type=f32)
    g1[...] = jnp.dot(bg1[...], Wp[...], preferred_element_type=f32)


def _scale_body(x_ref, d0_ref, d1_ref, xs0_ref, xs1_ref):
    x = x_ref[...]
    n0 = lax.rsqrt(jnp.maximum(d0_ref[...], 1.0))
    n1 = lax.rsqrt(jnp.maximum(d1_ref[...], 1.0))
    xs0_ref[...] = x * n0
    xs1_ref[...] = x * n1


_RB = 256
_NB = NPAD // _RB


def _score_body(y0_ref, y1_ref, d0_ref, d1_ref, M0_ref, M1_ref,
                k0_ref, k1_ref, w2_ref, sum_ref):
    i = pl.program_id(0)
    rows = lax.broadcasted_iota(i32, (_RB, 1), 0) + i * _RB
    mask = rows < N

    def part(y_ref, d_ref, M_ref, k_ref):
        yb = y_ref[...] * lax.rsqrt(jnp.maximum(d_ref[...], 1.0))
        a = jnp.tanh(jnp.dot(yb, M_ref[...], preferred_element_type=f32)
                     + k_ref[...])
        t = jnp.sum(a * w2_ref[...], axis=1, keepdims=True)
        return jnp.sum(jnp.where(mask, t, 0.0))

    s0 = part(y0_ref, d0_ref, M0_ref, k0_ref)
    s1 = part(y1_ref, d1_ref, M1_ref, k1_ref)

    row = lax.broadcasted_iota(i32, (8, 128), 0)
    lane = lax.broadcasted_iota(i32, (8, 128), 1)
    contrib = (jnp.where((row == 0) & (lane == 0), s0, 0.0)
               + jnp.where((row == 1) & (lane == 0), s1, 0.0))

    @pl.when(i == 0)
    def _():
        sum_ref[...] = jnp.zeros((8, 128), f32)

    sum_ref[...] += contrib


def _out_body(sum_ref, y0_ref, y1_ref, d0_ref, d1_ref, G0_ref, G1_ref,
              g0_ref, g1_ref, bp_ref, o_ref):
    srow = sum_ref[...]
    row = lax.broadcasted_iota(i32, (8, 128), 0)
    lane = lax.broadcasted_iota(i32, (8, 128), 1)
    w0 = jnp.sum(jnp.where((row == 0) & (lane == 0), srow, 0.0)) / N
    w1 = jnp.sum(jnp.where((row == 1) & (lane == 0), srow, 0.0)) / N
    m = jnp.maximum(w0, w1)
    e0 = jnp.exp(w0 - m)
    e1 = jnp.exp(w1 - m)
    b0 = e0 / (e0 + e1)
    b1 = e1 / (e0 + e1)
    yb0 = y0_ref[...] * lax.rsqrt(jnp.maximum(d0_ref[...], 1.0))
    yb1 = y1_ref[...] * lax.rsqrt(jnp.maximum(d1_ref[...], 1.0))
    p0 = jnp.dot(yb0, G0_ref[...], preferred_element_type=f32) + g0_ref[...]
    p1 = jnp.dot(yb1, G1_ref[...], preferred_element_type=f32) + g1_ref[...]
    o_ref[...] = b0 * p0 + b1 * p1 + bp_ref[...]


def _pad_edges(idx):
    pad = jnp.full((EPAD - E,), N, dtype=i32)
    return jnp.concatenate([idx, pad]).reshape(NTILES, CHUNKS, CHUNK)


def kernel(x, edge_index0, edge_index1, Wg0, bg0, Wg1, bg1,
           Ws1, bs1, Ws2, Wp, bp):
    src0 = _pad_edges(edge_index0[0])
    dst0 = _pad_edges(edge_index0[1])
    src1 = _pad_edges(edge_index1[0])
    dst1 = _pad_edges(edge_index1[1])
    x_pad = jnp.pad(x, ((0, NPAD - N), (0, 0)))
    z1 = jnp.zeros((ROWS_PER_TILE,), f32)
    z2 = jnp.zeros((ROWS_PER_TILE, IN), f32)

    ds0, di0, ds1, di1 = _hist_kernel(src0, dst0, src1, dst1, z1)

    M0, M1, k0, k1, G0, G1, g0, g1 = pl.pallas_call(
        _wfold_body,
        out_shape=[
            jax.ShapeDtypeStruct((IN, SEM_HID), f32),
            jax.ShapeDtypeStruct((IN, SEM_HID), f32),
            jax.ShapeDtypeStruct((1, SEM_HID), f32),
            jax.ShapeDtypeStruct((1, SEM_HID), f32),
            jax.ShapeDtypeStruct((IN, OUT), f32),
            jax.ShapeDtypeStruct((IN, OUT), f32),
            jax.ShapeDtypeStruct((1, OUT), f32),
            jax.ShapeDtypeStruct((1, OUT), f32),
        ],
    )(Wg0, Wg1, Ws1, bs1.reshape(1, SEM_HID), Wp,
      bg0.reshape(1, -1), bg1.reshape(1, -1))

    xs0, xs1 = pl.pallas_call(
        _scale_body,
        grid=(_NB,),
        in_specs=[
            pl.BlockSpec((_RB, IN), lambda i: (i, 0)),
            pl.BlockSpec((_RB, 1), lambda i: (i, 0)),
            pl.BlockSpec((_RB, 1), lambda i: (i, 0)),
        ],
        out_specs=[
            pl.BlockSpec((_RB, IN), lambda i: (i, 0)),
            pl.BlockSpec((_RB, IN), lambda i: (i, 0)),
        ],
        out_shape=[jax.ShapeDtypeStruct((NPAD, IN), f32)] * 2,
    )(x_pad, ds0.reshape(NPAD, 1), ds1.reshape(NPAD, 1))

    y0, y1 = _scatter_kernel(xs0, xs1, src0, dst0, src1, dst1, z2)

    di0c = di0.reshape(NPAD, 1)
    di1c = di1.reshape(NPAD, 1)
    w2r = Ws2.reshape(1, SEM_HID)

    full = lambda i: (0, 0)
    sums = pl.pallas_call(
        _score_body,
        grid=(_NB,),
        in_specs=[
            pl.BlockSpec((_RB, IN), lambda i: (i, 0)),
            pl.BlockSpec((_RB, IN), lambda i: (i, 0)),
            pl.BlockSpec((_RB, 1), lambda i: (i, 0)),
            pl.BlockSpec((_RB, 1), lambda i: (i, 0)),
            pl.BlockSpec((IN, SEM_HID), full),
            pl.BlockSpec((IN, SEM_HID), full),
            pl.BlockSpec((1, SEM_HID), full),
            pl.BlockSpec((1, SEM_HID), full),
            pl.BlockSpec((1, SEM_HID), full),
        ],
        out_specs=pl.BlockSpec((8, 128), full),
        out_shape=jax.ShapeDtypeStruct((8, 128), f32),
    )(y0, y1, di0c, di1c, M0, M1, k0, k1, w2r)

    outp = pl.pallas_call(
        _out_body,
        grid=(_NB,),
        in_specs=[
            pl.BlockSpec((8, 128), full),
            pl.BlockSpec((_RB, IN), lambda i: (i, 0)),
            pl.BlockSpec((_RB, IN), lambda i: (i, 0)),
            pl.BlockSpec((_RB, 1), lambda i: (i, 0)),
            pl.BlockSpec((_RB, 1), lambda i: (i, 0)),
            pl.BlockSpec((IN, OUT), full),
            pl.BlockSpec((IN, OUT), full),
            pl.BlockSpec((1, OUT), full),
            pl.BlockSpec((1, OUT), full),
            pl.BlockSpec((1, OUT), full),
        ],
        out_specs=pl.BlockSpec((_RB, OUT), lambda i: (i, 0)),
        out_shape=jax.ShapeDtypeStruct((NPAD, OUT), f32),
    )(sums, y0, y1, di0c, di1c, G0, G1, g0, g1, bp.reshape(1, OUT))

    return outp[:N]


# speedup vs baseline: 11.7444x; 1.3968x over previous
import functools
import jax
import jax.numpy as jnp
from jax import lax
from jax.experimental import pallas as pl
from jax.experimental.pallas import tpu as pltpu
from jax.experimental.pallas import tpu_sc as plsc

N = 10000
E = 320000
IN = 128
SEM_HID = 128
OUT = 64

NTILES = 16
ROWS_PER_TILE = 640
NPAD = NTILES * ROWS_PER_TILE
CHUNK = 112
NBUF = 3
GSTAGE = 6
STAGE = GSTAGE * NBUF
STAGES = 10
CHUNKS = STAGE * STAGES
GROUPS = CHUNKS // NBUF
EPT = CHUNKS * CHUNK
EPAD = NTILES * EPT

f32 = jnp.float32
i32 = jnp.int32

_MESH = plsc.VectorSubcoreMesh(core_axis_name="c", subcore_axis_name="s")


@functools.partial(
    pl.kernel,
    out_type=[jax.ShapeDtypeStruct((NPAD,), f32)] * 4,
    mesh=_MESH,
    scratch_types=[
        pltpu.VMEM((STAGE, CHUNK), i32),
        pltpu.VMEM((CHUNK,), f32),
        pltpu.VMEM_SHARED((NPAD,), f32),
        pltpu.VMEM_SHARED((NPAD,), f32),
    ],
)
def _hist_kernel(src0, dst0, src1, dst1, z1, ds0, di0, ds1, di1,
                 idx_v, ones_v, hsrc, hdst):
    c = lax.axis_index("c")
    s = lax.axis_index("s")
    for i in range(CHUNK // 16):
        ones_v[pl.ds(i * 16, 16)] = jnp.ones((16,), f32)
    sl = pl.ds(s * ROWS_PER_TILE, ROWS_PER_TILE)
    pltpu.sync_copy(z1, hsrc.at[sl])
    pltpu.sync_copy(z1, hdst.at[sl])
    plsc.subcore_barrier()

    def accumulate(edges_hbm, hist):
        def stage_body(st, carry):
            pltpu.sync_copy(edges_hbm.at[s * STAGES + st], idx_v)

            def body(j, carry2):
                pltpu.sync_copy(ones_v, hist.at[idx_v.at[j]], add=True)
                return carry2
            lax.fori_loop(0, STAGE, body, 0)
            return carry
        lax.fori_loop(0, STAGES, stage_body, 0)

    @pl.when(c == 0)
    def _():
        accumulate(src0, hsrc)
        accumulate(dst0, hdst)

    @pl.when(c == 1)
    def _():
        accumulate(src1, hsrc)
        accumulate(dst1, hdst)

    plsc.subcore_barrier()

    @pl.when(c == 0)
    def _():
        pltpu.sync_copy(hsrc.at[sl], ds0.at[sl])
        pltpu.sync_copy(hdst.at[sl], di0.at[sl])

    @pl.when(c == 1)
    def _():
        pltpu.sync_copy(hsrc.at[sl], ds1.at[sl])
        pltpu.sync_copy(hdst.at[sl], di1.at[sl])


@functools.partial(
    pl.kernel,
    out_type=[jax.ShapeDtypeStruct((NPAD, IN), f32)] * 2,
    mesh=_MESH,
    scratch_types=[
        pltpu.VMEM((STAGE, CHUNK), i32),
        pltpu.VMEM((STAGE, CHUNK), i32),
        [pltpu.VMEM((CHUNK, IN), f32)] * NBUF,
        [pltpu.SemaphoreType.DMA] * NBUF,
        [pltpu.SemaphoreType.DMA] * NBUF,
        pltpu.VMEM_SHARED((NPAD, IN), f32),
    ],
)
def _scatter_kernel(xs0, xs1, src0, dst0, src1, dst1, z2, y0, y1,
                    src_v, dst_v, bufs, gsems, ssems, ys):
    c = lax.axis_index("c")
    s = lax.axis_index("s")
    sl = pl.ds(s * ROWS_PER_TILE, ROWS_PER_TILE)
    pltpu.sync_copy(z2, ys.at[sl])
    plsc.subcore_barrier()

    def run(xs_hbm, src_hbm, dst_hbm):
        def group(gi, carry):
            at_stage = gi % GSTAGE == 0

            @pl.when(at_stage)
            def _():
                @pl.when(gi > 0)
                def _():
                    for b in range(NBUF):
                        pltpu.make_async_copy(
                            bufs[b], ys.at[dst_v.at[b]], ssems[b]).wait()
                st = gi // GSTAGE
                pltpu.sync_copy(src_hbm.at[s * STAGES + st], src_v)
                pltpu.sync_copy(dst_hbm.at[s * STAGES + st], dst_v)

            jj = (gi % GSTAGE) * NBUF
            for b in range(NBUF):
                @pl.when(jnp.logical_not(at_stage))
                def _():
                    pltpu.make_async_copy(
                        bufs[b], ys.at[dst_v.at[jj + b]], ssems[b]).wait()
                pltpu.async_copy(xs_hbm.at[src_v.at[jj + b]], bufs[b], gsems[b])
            for b in range(NBUF):
                pltpu.make_async_copy(
                    xs_hbm.at[src_v.at[jj + b]], bufs[b], gsems[b]).wait()
                pltpu.async_copy(bufs[b], ys.at[dst_v.at[jj + b]], ssems[b],
                                 add=True)
            return carry
        lax.fori_loop(0, GROUPS, group, 0)
        for b in range(NBUF):
            pltpu.make_async_copy(
                bufs[b], ys.at[dst_v.at[b]], ssems[b]).wait()

    @pl.when(c == 0)
    def _():
        run(xs0, src0, dst0)

    @pl.when(c == 1)
    def _():
        run(xs1, src1, dst1)

    plsc.subcore_barrier()

    @pl.when(c == 0)
    def _():
        pltpu.sync_copy(ys.at[sl], y0.at[sl])

    @pl.when(c == 1)
    def _():
        pltpu.sync_copy(ys.at[sl], y1.at[sl])


def _wfold_body(Wg0, Wg1, Ws1, bs1, Wp, bg0, bg1,
                M0, M1, k0, k1, G0, G1, g0, g1):
    M0[...] = jnp.dot(Wg0[...], Ws1[...], preferred_element_type=f32)
    M1[...] = jnp.dot(Wg1[...], Ws1[...], preferred_element_type=f32)
    k0[...] = jnp.dot(bg0[...], Ws1[...], preferred_element_type=f32) + bs1[...]
    k1[...] = jnp.dot(bg1[...], Ws1[...], preferred_element_type=f32) + bs1[...]
    G0[...] = jnp.dot(Wg0[...], Wp[...], preferred_element_type=f32)
    G1[...] = jnp.dot(Wg1[...], Wp[...], preferred_element_type=f32)
    g0[...] = jnp.dot(bg0[...], Wp[...], preferred_element_type=f32)
    g1[...] = jnp.dot(bg1[...], Wp[...], preferred_element_type=f32)


def _scale_body(x_ref, d0_ref, d1_ref, xs0_ref, xs1_ref):
    x = x_ref[...]
    n0 = lax.rsqrt(jnp.maximum(d0_ref[...], 1.0))
    n1 = lax.rsqrt(jnp.maximum(d1_ref[...], 1.0))
    xs0_ref[...] = x * n0
    xs1_ref[...] = x * n1


_RB = 256
_NB = NPAD // _RB


def _score_body(y0_ref, y1_ref, d0_ref, d1_ref, M0_ref, M1_ref,
                k0_ref, k1_ref, w2_ref, sum_ref):
    i = pl.program_id(0)
    rows = lax.broadcasted_iota(i32, (_RB, 1), 0) + i * _RB
    mask = rows < N

    def part(y_ref, d_ref, M_ref, k_ref):
        yb = y_ref[...] * lax.rsqrt(jnp.maximum(d_ref[...], 1.0))
        a = jnp.tanh(jnp.dot(yb, M_ref[...], preferred_element_type=f32)
                     + k_ref[...])
        t = jnp.sum(a * w2_ref[...], axis=1, keepdims=True)
        return jnp.sum(jnp.where(mask, t, 0.0))

    s0 = part(y0_ref, d0_ref, M0_ref, k0_ref)
    s1 = part(y1_ref, d1_ref, M1_ref, k1_ref)

    row = lax.broadcasted_iota(i32, (8, 128), 0)
    lane = lax.broadcasted_iota(i32, (8, 128), 1)
    contrib = (jnp.where((row == 0) & (lane == 0), s0, 0.0)
               + jnp.where((row == 1) & (lane == 0), s1, 0.0))

    @pl.when(i == 0)
    def _():
        sum_ref[...] = jnp.zeros((8, 128), f32)

    sum_ref[...] += contrib


def _out_body(sum_ref, y0_ref, y1_ref, d0_ref, d1_ref, G0_ref, G1_ref,
              g0_ref, g1_ref, bp_ref, o_ref):
    srow = sum_ref[...]
    row = lax.broadcasted_iota(i32, (8, 128), 0)
    lane = lax.broadcasted_iota(i32, (8, 128), 1)
    w0 = jnp.sum(jnp.where((row == 0) & (lane == 0), srow, 0.0)) / N
    w1 = jnp.sum(jnp.where((row == 1) & (lane == 0), srow, 0.0)) / N
    m = jnp.maximum(w0, w1)
    e0 = jnp.exp(w0 - m)
    e1 = jnp.exp(w1 - m)
    b0 = e0 / (e0 + e1)
    b1 = e1 / (e0 + e1)
    yb0 = y0_ref[...] * lax.rsqrt(jnp.maximum(d0_ref[...], 1.0))
    yb1 = y1_ref[...] * lax.rsqrt(jnp.maximum(d1_ref[...], 1.0))
    p0 = jnp.dot(yb0, G0_ref[...], preferred_element_type=f32) + g0_ref[...]
    p1 = jnp.dot(yb1, G1_ref[...], preferred_element_type=f32) + g1_ref[...]
    o_ref[...] = b0 * p0 + b1 * p1 + bp_ref[...]


def _pad_edges(idx):
    pad = jnp.full((EPAD - E,), N, dtype=i32)
    return jnp.concatenate([idx, pad]).reshape(NTILES * STAGES, STAGE, CHUNK)


def kernel(x, edge_index0, edge_index1, Wg0, bg0, Wg1, bg1,
           Ws1, bs1, Ws2, Wp, bp):
    src0 = _pad_edges(edge_index0[0])
    dst0 = _pad_edges(edge_index0[1])
    src1 = _pad_edges(edge_index1[0])
    dst1 = _pad_edges(edge_index1[1])
    x_pad = jnp.pad(x, ((0, NPAD - N), (0, 0)))
    z1 = jnp.zeros((ROWS_PER_TILE,), f32)
    z2 = jnp.zeros((ROWS_PER_TILE, IN), f32)

    ds0, di0, ds1, di1 = _hist_kernel(src0, dst0, src1, dst1, z1)

    M0, M1, k0, k1, G0, G1, g0, g1 = pl.pallas_call(
        _wfold_body,
        out_shape=[
            jax.ShapeDtypeStruct((IN, SEM_HID), f32),
            jax.ShapeDtypeStruct((IN, SEM_HID), f32),
            jax.ShapeDtypeStruct((1, SEM_HID), f32),
            jax.ShapeDtypeStruct((1, SEM_HID), f32),
            jax.ShapeDtypeStruct((IN, OUT), f32),
            jax.ShapeDtypeStruct((IN, OUT), f32),
            jax.ShapeDtypeStruct((1, OUT), f32),
            jax.ShapeDtypeStruct((1, OUT), f32),
        ],
    )(Wg0, Wg1, Ws1, bs1.reshape(1, SEM_HID), Wp,
      bg0.reshape(1, -1), bg1.reshape(1, -1))

    xs0, xs1 = pl.pallas_call(
        _scale_body,
        grid=(_NB,),
        in_specs=[
            pl.BlockSpec((_RB, IN), lambda i: (i, 0)),
            pl.BlockSpec((_RB, 1), lambda i: (i, 0)),
            pl.BlockSpec((_RB, 1), lambda i: (i, 0)),
        ],
        out_specs=[
            pl.BlockSpec((_RB, IN), lambda i: (i, 0)),
            pl.BlockSpec((_RB, IN), lambda i: (i, 0)),
        ],
        out_shape=[jax.ShapeDtypeStruct((NPAD, IN), f32)] * 2,
    )(x_pad, ds0.reshape(NPAD, 1), ds1.reshape(NPAD, 1))

    y0, y1 = _scatter_kernel(xs0, xs1, src0, dst0, src1, dst1, z2)

    di0c = di0.reshape(NPAD, 1)
    di1c = di1.reshape(NPAD, 1)
    w2r = Ws2.reshape(1, SEM_HID)

    full = lambda i: (0, 0)
    sums = pl.pallas_call(
        _score_body,
        grid=(_NB,),
        in_specs=[
            pl.BlockSpec((_RB, IN), lambda i: (i, 0)),
            pl.BlockSpec((_RB, IN), lambda i: (i, 0)),
            pl.BlockSpec((_RB, 1), lambda i: (i, 0)),
            pl.BlockSpec((_RB, 1), lambda i: (i, 0)),
            pl.BlockSpec((IN, SEM_HID), full),
            pl.BlockSpec((IN, SEM_HID), full),
            pl.BlockSpec((1, SEM_HID), full),
            pl.BlockSpec((1, SEM_HID), full),
            pl.BlockSpec((1, SEM_HID), full),
        ],
        out_specs=pl.BlockSpec((8, 128), full),
        out_shape=jax.ShapeDtypeStruct((8, 128), f32),
    )(y0, y1, di0c, di1c, M0, M1, k0, k1, w2r)

    outp = pl.pallas_call(
        _out_body,
        grid=(_NB,),
        in_specs=[
            pl.BlockSpec((8, 128), full),
            pl.BlockSpec((_RB, IN), lambda i: (i, 0)),
            pl.BlockSpec((_RB, IN), lambda i: (i, 0)),
            pl.BlockSpec((_RB, 1), lambda i: (i, 0)),
            pl.BlockSpec((_RB, 1), lambda i: (i, 0)),
            pl.BlockSpec((IN, OUT), full),
            pl.BlockSpec((IN, OUT), full),
            pl.BlockSpec((1, OUT), full),
            pl.BlockSpec((1, OUT), full),
            pl.BlockSpec((1, OUT), full),
        ],
        out_specs=pl.BlockSpec((_RB, OUT), lambda i: (i, 0)),
        out_shape=jax.ShapeDtypeStruct((NPAD, OUT), f32),
    )(sums, y0, y1, di0c, di1c, G0, G1, g0, g1, bp.reshape(1, OUT))

    return outp[:N]


# speedup vs baseline: 12.2465x; 1.0427x over previous
import functools
import jax
import jax.numpy as jnp
from jax import lax
from jax.experimental import pallas as pl
from jax.experimental.pallas import tpu as pltpu
from jax.experimental.pallas import tpu_sc as plsc

N = 10000
E = 320000
IN = 128
SEM_HID = 128
OUT = 64

NTILES = 16
ROWS_PER_TILE = 640
NPAD = NTILES * ROWS_PER_TILE
CHUNK = 112
NBUF = 3
GSTAGE = 6
STAGE = GSTAGE * NBUF
STAGES = 10
CHUNKS = STAGE * STAGES
GROUPS = CHUNKS // NBUF
EPT = CHUNKS * CHUNK
EPAD = NTILES * EPT

f32 = jnp.float32
i32 = jnp.int32

_MESH = plsc.VectorSubcoreMesh(core_axis_name="c", subcore_axis_name="s")


@functools.partial(
    pl.kernel,
    out_type=[jax.ShapeDtypeStruct((NPAD,), f32)] * 4,
    mesh=_MESH,
    scratch_types=[
        pltpu.VMEM((STAGE, CHUNK), i32),
        pltpu.VMEM((CHUNK,), f32),
        pltpu.VMEM_SHARED((NPAD,), f32),
        pltpu.VMEM_SHARED((NPAD,), f32),
        pltpu.SemaphoreType.DMA,
    ],
)
def _hist_kernel(src0, dst0, src1, dst1, z1, ds0, di0, ds1, di1,
                 idx_v, ones_v, hsrc, hdst, hsem):
    c = lax.axis_index("c")
    s = lax.axis_index("s")
    for i in range(CHUNK // 16):
        ones_v[pl.ds(i * 16, 16)] = jnp.ones((16,), f32)
    sl = pl.ds(s * ROWS_PER_TILE, ROWS_PER_TILE)
    pltpu.sync_copy(z1, hsrc.at[sl])
    pltpu.sync_copy(z1, hdst.at[sl])
    plsc.subcore_barrier()

    def accumulate(edges_hbm, hist, sem):
        def stage_body(st, carry):
            pltpu.sync_copy(edges_hbm.at[s * STAGES + st], idx_v)

            def body(j, carry2):
                pltpu.async_copy(ones_v, hist.at[idx_v.at[j]], sem, add=True)
                return carry2
            lax.fori_loop(0, STAGE, body, 0)

            def drain(j, carry2):
                pltpu.make_async_copy(ones_v, hist.at[idx_v.at[0]], sem).wait()
                return carry2
            lax.fori_loop(0, STAGE, drain, 0)
            return carry
        lax.fori_loop(0, STAGES, stage_body, 0)

    @pl.when(c == 0)
    def _():
        accumulate(src0, hsrc, hsem)
        accumulate(dst0, hdst, hsem)

    @pl.when(c == 1)
    def _():
        accumulate(src1, hsrc, hsem)
        accumulate(dst1, hdst, hsem)

    plsc.subcore_barrier()

    @pl.when(c == 0)
    def _():
        pltpu.sync_copy(hsrc.at[sl], ds0.at[sl])
        pltpu.sync_copy(hdst.at[sl], di0.at[sl])

    @pl.when(c == 1)
    def _():
        pltpu.sync_copy(hsrc.at[sl], ds1.at[sl])
        pltpu.sync_copy(hdst.at[sl], di1.at[sl])


@functools.partial(
    pl.kernel,
    out_type=[jax.ShapeDtypeStruct((NPAD, IN), f32)] * 2,
    mesh=_MESH,
    scratch_types=[
        pltpu.VMEM((STAGE, CHUNK), i32),
        pltpu.VMEM((STAGE, CHUNK), i32),
        [pltpu.VMEM((CHUNK, IN), f32)] * NBUF,
        [pltpu.SemaphoreType.DMA] * NBUF,
        [pltpu.SemaphoreType.DMA] * NBUF,
        pltpu.VMEM_SHARED((NPAD, IN), f32),
    ],
)
def _scatter_kernel(xs0, xs1, src0, dst0, src1, dst1, z2, y0, y1,
                    src_v, dst_v, bufs, gsems, ssems, ys):
    c = lax.axis_index("c")
    s = lax.axis_index("s")
    sl = pl.ds(s * ROWS_PER_TILE, ROWS_PER_TILE)
    pltpu.sync_copy(z2, ys.at[sl])
    plsc.subcore_barrier()

    def run(xs_hbm, src_hbm, dst_hbm):
        def group(gi, carry):
            at_stage = gi % GSTAGE == 0

            @pl.when(at_stage)
            def _():
                @pl.when(gi > 0)
                def _():
                    for b in range(NBUF):
                        pltpu.make_async_copy(
                            bufs[b], ys.at[dst_v.at[b]], ssems[b]).wait()
                st = gi // GSTAGE
                pltpu.sync_copy(src_hbm.at[s * STAGES + st], src_v)
                pltpu.sync_copy(dst_hbm.at[s * STAGES + st], dst_v)

            jj = (gi % GSTAGE) * NBUF
            for b in range(NBUF):
                @pl.when(jnp.logical_not(at_stage))
                def _():
                    pltpu.make_async_copy(
                        bufs[b], ys.at[dst_v.at[jj + b]], ssems[b]).wait()
                pltpu.async_copy(xs_hbm.at[src_v.at[jj + b]], bufs[b], gsems[b])
            for b in range(NBUF):
                pltpu.make_async_copy(
                    xs_hbm.at[src_v.at[jj + b]], bufs[b], gsems[b]).wait()
                pltpu.async_copy(bufs[b], ys.at[dst_v.at[jj + b]], ssems[b],
                                 add=True)
            return carry
        lax.fori_loop(0, GROUPS, group, 0)
        for b in range(NBUF):
            pltpu.make_async_copy(
                bufs[b], ys.at[dst_v.at[b]], ssems[b]).wait()

    @pl.when(c == 0)
    def _():
        run(xs0, src0, dst0)

    @pl.when(c == 1)
    def _():
        run(xs1, src1, dst1)

    plsc.subcore_barrier()

    @pl.when(c == 0)
    def _():
        pltpu.sync_copy(ys.at[sl], y0.at[sl])

    @pl.when(c == 1)
    def _():
        pltpu.sync_copy(ys.at[sl], y1.at[sl])


def _scale_body(x_ref, d0_ref, d1_ref, xs0_ref, xs1_ref):
    x = x_ref[...]
    n0 = lax.rsqrt(jnp.maximum(d0_ref[...], 1.0))
    n1 = lax.rsqrt(jnp.maximum(d1_ref[...], 1.0))
    xs0_ref[...] = x * n0
    xs1_ref[...] = x * n1


_RB = 256
_NB = NPAD // _RB


def _dense_body(y0_ref, y1_ref, d0_ref, d1_ref, Wg0_ref, Wg1_ref, Ws1_ref,
                bs1_ref, w2_ref, Wp_ref, bg0_ref, bg1_ref, bp_ref, o_ref,
                acc, M0s, M1s, k0s, k1s, G0s, G1s, g0s, g1s):
    p = pl.program_id(0)
    i = pl.program_id(1)

    @pl.when((p == 0) & (i == 0))
    def _():
        M0s[...] = jnp.dot(Wg0_ref[...], Ws1_ref[...],
                           preferred_element_type=f32)
        M1s[...] = jnp.dot(Wg1_ref[...], Ws1_ref[...],
                           preferred_element_type=f32)
        k0s[...] = jnp.dot(bg0_ref[...], Ws1_ref[...],
                           preferred_element_type=f32) + bs1_ref[...]
        k1s[...] = jnp.dot(bg1_ref[...], Ws1_ref[...],
                           preferred_element_type=f32) + bs1_ref[...]
        G0s[...] = jnp.dot(Wg0_ref[...], Wp_ref[...],
                           preferred_element_type=f32)
        G1s[...] = jnp.dot(Wg1_ref[...], Wp_ref[...],
                           preferred_element_type=f32)
        g0s[...] = jnp.dot(bg0_ref[...], Wp_ref[...],
                           preferred_element_type=f32)
        g1s[...] = jnp.dot(bg1_ref[...], Wp_ref[...],
                           preferred_element_type=f32)
        acc[...] = jnp.zeros((8, 128), f32)

    yb0 = y0_ref[...] * lax.rsqrt(jnp.maximum(d0_ref[...], 1.0))
    yb1 = y1_ref[...] * lax.rsqrt(jnp.maximum(d1_ref[...], 1.0))
    row = lax.broadcasted_iota(i32, (8, 128), 0)
    lane = lax.broadcasted_iota(i32, (8, 128), 1)

    @pl.when(p == 0)
    def _():
        rows = lax.broadcasted_iota(i32, (_RB, 1), 0) + i * _RB
        mask = rows < N

        def part(yb, M_ref, k_ref):
            a = jnp.tanh(jnp.dot(yb, M_ref[...], preferred_element_type=f32)
                         + k_ref[...])
            t = jnp.sum(a * w2_ref[...], axis=1, keepdims=True)
            return jnp.sum(jnp.where(mask, t, 0.0))

        s0 = part(yb0, M0s, k0s)
        s1 = part(yb1, M1s, k1s)
        acc[...] += (jnp.where((row == 0) & (lane == 0), s0, 0.0)
                     + jnp.where((row == 1) & (lane == 0), s1, 0.0))

    @pl.when(p == 1)
    def _():
        srow = acc[...]
        w0 = jnp.sum(jnp.where((row == 0) & (lane == 0), srow, 0.0)) / N
        w1 = jnp.sum(jnp.where((row == 1) & (lane == 0), srow, 0.0)) / N
        m = jnp.maximum(w0, w1)
        e0 = jnp.exp(w0 - m)
        e1 = jnp.exp(w1 - m)
        b0 = e0 / (e0 + e1)
        b1 = e1 / (e0 + e1)
        p0 = jnp.dot(yb0, G0s[...], preferred_element_type=f32) + g0s[...]
        p1 = jnp.dot(yb1, G1s[...], preferred_element_type=f32) + g1s[...]
        o_ref[...] = b0 * p0 + b1 * p1 + bp_ref[...]


def _pad_edges(idx):
    pad = jnp.full((EPAD - E,), N, dtype=i32)
    return jnp.concatenate([idx, pad]).reshape(NTILES * STAGES, STAGE, CHUNK)


def kernel(x, edge_index0, edge_index1, Wg0, bg0, Wg1, bg1,
           Ws1, bs1, Ws2, Wp, bp):
    src0 = _pad_edges(edge_index0[0])
    dst0 = _pad_edges(edge_index0[1])
    src1 = _pad_edges(edge_index1[0])
    dst1 = _pad_edges(edge_index1[1])
    x_pad = jnp.pad(x, ((0, NPAD - N), (0, 0)))
    z1 = jnp.zeros((ROWS_PER_TILE,), f32)
    z2 = jnp.zeros((ROWS_PER_TILE, IN), f32)

    ds0, di0, ds1, di1 = _hist_kernel(src0, dst0, src1, dst1, z1)

    xs0, xs1 = pl.pallas_call(
        _scale_body,
        grid=(_NB,),
        in_specs=[
            pl.BlockSpec((_RB, IN), lambda i: (i, 0)),
            pl.BlockSpec((_RB, 1), lambda i: (i, 0)),
            pl.BlockSpec((_RB, 1), lambda i: (i, 0)),
        ],
        out_specs=[
            pl.BlockSpec((_RB, IN), lambda i: (i, 0)),
            pl.BlockSpec((_RB, IN), lambda i: (i, 0)),
        ],
        out_shape=[jax.ShapeDtypeStruct((NPAD, IN), f32)] * 2,
    )(x_pad, ds0.reshape(NPAD, 1), ds1.reshape(NPAD, 1))

    y0, y1 = _scatter_kernel(xs0, xs1, src0, dst0, src1, dst1, z2)

    di0c = di0.reshape(NPAD, 1)
    di1c = di1.reshape(NPAD, 1)

    DH = Wg0.shape[1]
    blk = lambda p, i: (i, 0)
    full = lambda p, i: (0, 0)
    outp = pl.pallas_call(
        _dense_body,
        grid=(2, _NB),
        in_specs=[
            pl.BlockSpec((_RB, IN), blk),
            pl.BlockSpec((_RB, IN), blk),
            pl.BlockSpec((_RB, 1), blk),
            pl.BlockSpec((_RB, 1), blk),
            pl.BlockSpec((IN, DH), full),
            pl.BlockSpec((IN, DH), full),
            pl.BlockSpec((DH, SEM_HID), full),
            pl.BlockSpec((1, SEM_HID), full),
            pl.BlockSpec((1, SEM_HID), full),
            pl.BlockSpec((DH, OUT), full),
            pl.BlockSpec((1, DH), full),
            pl.BlockSpec((1, DH), full),
            pl.BlockSpec((1, OUT), full),
        ],
        out_specs=pl.BlockSpec((_RB, OUT), blk),
        out_shape=jax.ShapeDtypeStruct((NPAD, OUT), f32),
        scratch_shapes=[
            pltpu.VMEM((8, 128), f32),
            pltpu.VMEM((IN, SEM_HID), f32),
            pltpu.VMEM((IN, SEM_HID), f32),
            pltpu.VMEM((1, SEM_HID), f32),
            pltpu.VMEM((1, SEM_HID), f32),
            pltpu.VMEM((IN, OUT), f32),
            pltpu.VMEM((IN, OUT), f32),
            pltpu.VMEM((1, OUT), f32),
            pltpu.VMEM((1, OUT), f32),
        ],
    )(y0, y1, di0c, di1c, Wg0, Wg1, Ws1,
      bs1.reshape(1, SEM_HID), Ws2.reshape(1, SEM_HID), Wp,
      bg0.reshape(1, DH), bg1.reshape(1, DH), bp.reshape(1, OUT))

    return outp[:N]
